# baseline (speedup 1.0000x reference)
import functools
import jax
import jax.numpy as jnp
from jax import lax
from jax.experimental import pallas as pl
from jax.experimental.pallas import tpu as pltpu
from jax.experimental.pallas import tpu_sc as plsc

_N = 50000
_D = 256
_S = 1000
_E = 100000

_SP = 1024
_NT = 16
_EC = 6272
_EP = _NT * _EC
_NGRP = _EC // 16
_COLS = _SP // _NT

_RC = 1568
_GBP = 32 * _RC
_SUB = 112
_NSUB = _RC // _SUB


def _mesh():
    return plsc.VectorSubcoreMesh(core_axis_name="c", subcore_axis_name="s")


def _fused_body(xt, nsx, bx, winner, bidx,
                a1w1, a1b1, a1w2, a1b2, a1w3, a1b3,
                a2w1a, a2w1b, a2b1, a2w2, a2b2, a2w3, a2b3,
                gw, gbias, out_ref):
    f32 = jnp.float32
    w1 = a1w1[...]
    h = jnp.maximum(
        jnp.dot(xt[...], w1[:_D], preferred_element_type=f32)
        + jnp.dot(nsx[...], w1[_D:], preferred_element_type=f32)
        + a1b1[...], 0.0)
    h = jnp.maximum(jnp.dot(h, a1w2[...], preferred_element_type=f32) + a1b2[...], 0.0)
    u = jnp.dot(h, a1w3[...], preferred_element_type=f32) + a1b3[...]
    rows = lax.broadcasted_iota(jnp.int32, (_SP, _D), 0)
    u = jnp.where(rows < _S, u, 0.0)

    bidxv = bidx[...]
    idx_u = jnp.minimum(bidxv, _SP - 1)
    cols = lax.broadcasted_iota(jnp.int32, (_SP, _SP), 1)
    onehot = (idx_u == cols).astype(f32)
    ug = jnp.dot(onehot, u, preferred_element_type=f32)
    b = jnp.where(bidxv < _S, ug, bx[...])
    h = jnp.maximum(
        jnp.dot(u, a2w1a[...], preferred_element_type=f32)
        + jnp.dot(b, a2w1b[...], preferred_element_type=f32)
        + a2b1[...], 0.0)
    h = jnp.maximum(jnp.dot(h, a2w2[...], preferred_element_type=f32) + a2b2[...], 0.0)
    cand = jnp.dot(h, a2w3[...], preferred_element_type=f32) + a2b3[...]
    x3 = jnp.where(winner[...] >= 0, cand, u)
    out_ref[...] = jnp.dot(x3, gw[...], preferred_element_type=f32) + gbias[...]


def _route_body(sei0_hbm, sei1_hbm, x_hbm, winner_hbm, bidx_hbm, bx_hbm,
                chunk_v, wloc_v, wall_v, wfin_v, bidx_v, rows_v, wall_sh, sem):
    c = lax.axis_index("c")
    sid = lax.axis_index("s")

    @pl.when(c == 0)
    def _core0():
        base = sid * _EC
        pltpu.sync_copy(sei0_hbm.at[pl.ds(base, _EC)], chunk_v)

        neg1 = jnp.full((16,), -1, jnp.int32)

        def _init(i, carry):
            for u in range(4):
                wloc_v[pl.ds((i * 4 + u) * 16, 16)] = neg1
            return carry

        lax.fori_loop(0, _SP // 64, _init, 0)

        lanes = lax.iota(jnp.int32, 16)

        def _scatter(g, carry):
            for u in range(4):
                off = (g * 4 + u) * 16
                idx = chunk_v[pl.ds(off, 16)]
                jv = lanes + (base + off)
                plsc.store_scatter(wloc_v, [idx], jv, mask=idx < _S)
            return carry

        lax.fori_loop(0, _NGRP // 4, _scatter, 0)

        pltpu.sync_copy(wloc_v, wall_sh.at[sid])
        plsc.subcore_barrier()
        pltpu.sync_copy(wall_sh, wall_v)

        col = sid * _COLS
        for h in range(_COLS // 16):
            acc = wall_v[0, pl.ds(col + h * 16, 16)]
            for r in range(1, _NT):
                acc = jnp.maximum(acc, wall_v[r, pl.ds(col + h * 16, 16)])
            wfin_v[pl.ds(h * 16, 16)] = acc

        pltpu.sync_copy(wfin_v, winner_hbm.at[pl.ds(col, _COLS)])

        for h in range(_COLS // 16):
            v = wfin_v[pl.ds(h * 16, 16)]
            wfin_v[pl.ds(h * 16, 16)] = jnp.maximum(v, 0)

        pltpu.async_copy(sei1_hbm.at[wfin_v], bidx_v, sem).wait()
        pltpu.sync_copy(bidx_v, bidx_hbm.at[pl.ds(col, _COLS)])
        pltpu.async_copy(x_hbm.at[bidx_v], rows_v, sem).wait()
        pltpu.sync_copy(rows_v, bx_hbm.at[pl.ds(col, _COLS)])


def _expand_body(g_hbm, gb_hbm, out_hbm, idx_v,
                 rows0, rows1, rows2, rows3, sem0, sem1, sem2, sem3):
    c = lax.axis_index("c")
    sid = lax.axis_index("s")
    wid = sid * 2 + c
    base = wid * _RC
    rows = (rows0, rows1, rows2, rows3)
    sems = (sem0, sem1, sem2, sem3)
    pltpu.sync_copy(gb_hbm.at[pl.ds(base, _RC)], idx_v)

    def _gather(k):
        return pltpu.async_copy(
            g_hbm.at[idx_v.at[pl.ds(k * _SUB, _SUB)]], rows[k % 4], sems[k % 4])

    gd = {k: _gather(k) for k in range(3)}
    sd = {}
    for k in range(_NSUB):
        gd[k].wait()
        start = base + k * _SUB
        buf = rows[k % 4]
        if k <= _NSUB - 3:
            sd[k] = pltpu.async_copy(buf, out_hbm.at[pl.ds(start, _SUB)],
                                     sems[k % 4])
        elif k == _NSUB - 2:
            @pl.when(start + _SUB <= _N)
            def _full():
                pltpu.sync_copy(buf, out_hbm.at[pl.ds(start, _SUB)])

            @pl.when(start + _SUB > _N)
            def _tail():
                for t in range(_SUB // 8):
                    @pl.when(start + t * 8 + 8 <= _N)
                    def _piece():
                        pltpu.sync_copy(
                            buf.at[pl.ds(t * 8, 8)],
                            out_hbm.at[pl.ds(start + t * 8, 8)])
        else:
            @pl.when(start + _SUB <= _N)
            def _last():
                pltpu.sync_copy(buf, out_hbm.at[pl.ds(start, _SUB)])
        if k >= 1 and (k - 1) in sd:
            sd.pop(k - 1).wait()
        if k + 3 < _NSUB:
            gd[k + 3] = _gather(k + 3)
    for k in list(sd):
        sd.pop(k).wait()


_BLK = 256
_NBLK = _GBP // _BLK


def _expand_tc_body(w_sm, last_sm, gbv, g_ref, out_ref):
    b = pl.program_id(0)
    f32 = jnp.float32
    w = w_sm[b]
    last = last_sm[b]
    wstart = pl.multiple_of((jnp.minimum(w, _SP - _BLK) // 8) * 8, 8)
    gbb = gbv[...].reshape(_BLK, 1)
    span_ok = (last - wstart) < _BLK

    @pl.when(span_ok)
    def _fast():
        cols = lax.broadcasted_iota(jnp.int32, (_BLK, _BLK), 1)
        onehot = (gbb - wstart == cols).astype(f32)
        win = g_ref[pl.ds(wstart, _BLK), :]
        out_ref[...] = jnp.dot(onehot, win, preferred_element_type=f32,
                               precision=lax.Precision.HIGHEST)

    @pl.when(jnp.logical_not(span_ok))
    def _slow():
        cols = lax.broadcasted_iota(jnp.int32, (_BLK, _SP), 1)
        onehot = (gbb == cols).astype(f32)
        out_ref[...] = jnp.dot(onehot, g_ref[...], preferred_element_type=f32,
                               precision=lax.Precision.HIGHEST)


def kernel(x, new_supernode_x, supernode_edge_index, supernode_idx,
           graph_batch, A1_W1, A1_b1, A1_W2, A1_b2, A1_W3, A1_b3,
           A2_W1, A2_b1, A2_W2, A2_b2, A2_W3, A2_b3,
           gat_W, gat_a_src, gat_a_dst, gat_b):
    f32, i32 = jnp.float32, jnp.int32

    sei0 = supernode_edge_index[0]
    sei1 = supernode_edge_index[1]
    sei0p = jnp.concatenate([sei0, jnp.full((_EP - _E,), _N, i32)])
    sei1p = jnp.concatenate([sei1, jnp.zeros((_EP - _E,), i32)])
    gbp = jnp.concatenate(
        [graph_batch, jnp.broadcast_to(graph_batch[-1], (_GBP - _N,))])

    route = functools.partial(
        pl.kernel,
        out_type=[
            jax.ShapeDtypeStruct((_SP,), i32),
            jax.ShapeDtypeStruct((_SP,), i32),
            jax.ShapeDtypeStruct((_SP, _D), f32),
        ],
        mesh=_mesh(),
        compiler_params=pltpu.CompilerParams(needs_layout_passes=False),
        scratch_types=[
            pltpu.VMEM((_EC,), i32),
            pltpu.VMEM((_SP,), i32),
            pltpu.VMEM((_NT, _SP), i32),
            pltpu.VMEM((_COLS,), i32),
            pltpu.VMEM((_COLS,), i32),
            pltpu.VMEM((_COLS, _D), f32),
            pltpu.VMEM_SHARED((_NT, _SP), i32),
            pltpu.SemaphoreType.DMA,
        ],
    )(_route_body)
    winner, bidx, bx = route(sei0p, sei1p, x)

    x_top = lax.slice(x, (0, 0), (_SP, _D))
    nsx_p = jnp.concatenate(
        [new_supernode_x, jnp.zeros((_SP - _S, _D), f32)], axis=0)
    g = pl.pallas_call(
        _fused_body,
        out_shape=jax.ShapeDtypeStruct((_SP, _D), f32),
    )(x_top, nsx_p, bx, winner.reshape(_SP, 1), bidx.reshape(_SP, 1),
      A1_W1, A1_b1.reshape(1, -1), A1_W2, A1_b2.reshape(1, -1),
      A1_W3, A1_b3.reshape(1, -1),
      A2_W1[:_D], A2_W1[_D:], A2_b1.reshape(1, -1),
      A2_W2, A2_b2.reshape(1, -1), A2_W3, A2_b3.reshape(1, -1),
      gat_W, gat_b.reshape(1, -1))

    w_arr = lax.slice(gbp, (0,), (_GBP,), (_BLK,))
    l_arr = lax.slice(gbp, (_BLK - 1,), (_GBP,), (_BLK,))
    gb_cols = gbp.reshape(_NBLK, _BLK, 1)
    return pl.pallas_call(
        _expand_tc_body,
        grid_spec=pltpu.PrefetchScalarGridSpec(
            num_scalar_prefetch=2,
            grid=(_NBLK,),
            in_specs=[
                pl.BlockSpec((1, _BLK, 1), lambda b, *_: (b, 0, 0)),
                pl.BlockSpec((_SP, _D), lambda b, *_: (0, 0)),
            ],
            out_specs=pl.BlockSpec((_BLK, _D), lambda b, *_: (b, 0)),
        ),
        out_shape=jax.ShapeDtypeStruct((_N, _D), f32),
    )(w_arr, l_arr, gb_cols, g)


# speedup vs baseline: 1.6252x; 1.6252x over previous
import functools
import jax
import jax.numpy as jnp
from jax import lax
from jax.experimental import pallas as pl
from jax.experimental.pallas import tpu as pltpu
from jax.experimental.pallas import tpu_sc as plsc

_N = 50000
_D = 256
_S = 1000
_E = 100000

_SP = 1024
_NT = 16
_EC = 6272
_EP = _NT * _EC
_NGRP = _EC // 16
_COLS = _SP // _NT

_RC = 1568
_GBP = 32 * _RC
_SUB = 112
_NSUB = _RC // _SUB


def _mesh():
    return plsc.VectorSubcoreMesh(core_axis_name="c", subcore_axis_name="s")


def _fused_body(xt, nsx, bx, winner, bidx,
                a1w1, a1b1, a1w2, a1b2, a1w3, a1b3,
                a2w1a, a2w1b, a2b1, a2w2, a2b2, a2w3, a2b3,
                gw, gbias, out_ref):
    f32 = jnp.float32
    w1 = a1w1[...]
    h = jnp.maximum(
        jnp.dot(xt[...], w1[:_D], preferred_element_type=f32)
        + jnp.dot(nsx[...], w1[_D:], preferred_element_type=f32)
        + a1b1[...], 0.0)
    h = jnp.maximum(jnp.dot(h, a1w2[...], preferred_element_type=f32) + a1b2[...], 0.0)
    u = jnp.dot(h, a1w3[...], preferred_element_type=f32) + a1b3[...]
    rows = lax.broadcasted_iota(jnp.int32, (_SP, _D), 0)
    u = jnp.where(rows < _S, u, 0.0)

    bidxv = bidx[...]
    idx_u = jnp.minimum(bidxv, _SP - 1)
    cols = lax.broadcasted_iota(jnp.int32, (_SP, _SP), 1)
    onehot = (idx_u == cols).astype(f32)
    ug = jnp.dot(onehot, u, preferred_element_type=f32)
    b = jnp.where(bidxv < _S, ug, bx[...])
    h = jnp.maximum(
        jnp.dot(u, a2w1a[...], preferred_element_type=f32)
        + jnp.dot(b, a2w1b[...], preferred_element_type=f32)
        + a2b1[...], 0.0)
    h = jnp.maximum(jnp.dot(h, a2w2[...], preferred_element_type=f32) + a2b2[...], 0.0)
    cand = jnp.dot(h, a2w3[...], preferred_element_type=f32) + a2b3[...]
    x3 = jnp.where(winner[...] >= 0, cand, u)
    out_ref[...] = jnp.dot(x3, gw[...], preferred_element_type=f32) + gbias[...]


def _route_body(sei0_hbm, sei1_hbm, x_hbm, winner_hbm, bidx_hbm, bx_hbm,
                chunk_v, wloc_v, wall_v, wfin_v, bidx_v, rows_v, wall_sh, sem):
    c = lax.axis_index("c")
    sid = lax.axis_index("s")

    @pl.when(c == 0)
    def _core0():
        base = sid * _EC
        pltpu.sync_copy(sei0_hbm.at[pl.ds(base, _EC)], chunk_v)

        neg1 = jnp.full((16,), -1, jnp.int32)

        def _init(i, carry):
            for u in range(4):
                wloc_v[pl.ds((i * 4 + u) * 16, 16)] = neg1
            return carry

        lax.fori_loop(0, _SP // 64, _init, 0)

        lanes = lax.iota(jnp.int32, 16)

        def _scatter(g, carry):
            for u in range(4):
                off = (g * 4 + u) * 16
                idx = chunk_v[pl.ds(off, 16)]
                jv = lanes + (base + off)
                plsc.store_scatter(wloc_v, [idx], jv, mask=idx < _S)
            return carry

        lax.fori_loop(0, _NGRP // 4, _scatter, 0)

        pltpu.sync_copy(wloc_v, wall_sh.at[sid])
        plsc.subcore_barrier()
        pltpu.sync_copy(wall_sh, wall_v)

        col = sid * _COLS
        for h in range(_COLS // 16):
            acc = wall_v[0, pl.ds(col + h * 16, 16)]
            for r in range(1, _NT):
                acc = jnp.maximum(acc, wall_v[r, pl.ds(col + h * 16, 16)])
            wfin_v[pl.ds(h * 16, 16)] = acc

        pltpu.sync_copy(wfin_v, winner_hbm.at[pl.ds(col, _COLS)])

        for h in range(_COLS // 16):
            v = wfin_v[pl.ds(h * 16, 16)]
            wfin_v[pl.ds(h * 16, 16)] = jnp.maximum(v, 0)

        pltpu.async_copy(sei1_hbm.at[wfin_v], bidx_v, sem).wait()
        pltpu.sync_copy(bidx_v, bidx_hbm.at[pl.ds(col, _COLS)])
        pltpu.async_copy(x_hbm.at[bidx_v], rows_v, sem).wait()
        pltpu.sync_copy(rows_v, bx_hbm.at[pl.ds(col, _COLS)])


def _expand_body(g_hbm, gb_hbm, out_hbm, idx_v,
                 rows0, rows1, rows2, rows3, sem0, sem1, sem2, sem3):
    c = lax.axis_index("c")
    sid = lax.axis_index("s")
    wid = sid * 2 + c
    base = wid * _RC
    rows = (rows0, rows1, rows2, rows3)
    sems = (sem0, sem1, sem2, sem3)
    pltpu.sync_copy(gb_hbm.at[pl.ds(base, _RC)], idx_v)

    def _gather(k):
        return pltpu.async_copy(
            g_hbm.at[idx_v.at[pl.ds(k * _SUB, _SUB)]], rows[k % 4], sems[k % 4])

    gd = {k: _gather(k) for k in range(3)}
    sd = {}
    for k in range(_NSUB):
        gd[k].wait()
        start = base + k * _SUB
        buf = rows[k % 4]
        if k <= _NSUB - 3:
            sd[k] = pltpu.async_copy(buf, out_hbm.at[pl.ds(start, _SUB)],
                                     sems[k % 4])
        elif k == _NSUB - 2:
            @pl.when(start + _SUB <= _N)
            def _full():
                pltpu.sync_copy(buf, out_hbm.at[pl.ds(start, _SUB)])

            @pl.when(start + _SUB > _N)
            def _tail():
                for t in range(_SUB // 8):
                    @pl.when(start + t * 8 + 8 <= _N)
                    def _piece():
                        pltpu.sync_copy(
                            buf.at[pl.ds(t * 8, 8)],
                            out_hbm.at[pl.ds(start + t * 8, 8)])
        else:
            @pl.when(start + _SUB <= _N)
            def _last():
                pltpu.sync_copy(buf, out_hbm.at[pl.ds(start, _SUB)])
        if k >= 1 and (k - 1) in sd:
            sd.pop(k - 1).wait()
        if k + 3 < _NSUB:
            gd[k + 3] = _gather(k + 3)
    for k in list(sd):
        sd.pop(k).wait()


_BLK = 1024
_NBLK = _GBP // _BLK
_WINB = 128


def _expand_tc_body(w_sm, last_sm, gbv, g_ref, out_ref):
    b = pl.program_id(0)
    f32 = jnp.float32
    w = w_sm[b]
    last = last_sm[b]
    wstart = pl.multiple_of((jnp.minimum(w, _SP - _WINB) // 8) * 8, 8)
    gbb = gbv[...].reshape(_BLK, 1)
    span_ok = (last - wstart) < _WINB

    @pl.when(span_ok)
    def _fast():
        cols = lax.broadcasted_iota(jnp.int32, (_BLK, _WINB), 1)
        onehot = (gbb - wstart == cols).astype(f32)
        win = g_ref[pl.ds(wstart, _WINB), :]
        out_ref[...] = jnp.dot(onehot, win, preferred_element_type=f32,
                               precision=lax.Precision.HIGHEST)

    @pl.when(jnp.logical_not(span_ok))
    def _slow():
        cols = lax.broadcasted_iota(jnp.int32, (_BLK, _SP), 1)
        onehot = (gbb == cols).astype(f32)
        out_ref[...] = jnp.dot(onehot, g_ref[...], preferred_element_type=f32,
                               precision=lax.Precision.HIGHEST)


def kernel(x, new_supernode_x, supernode_edge_index, supernode_idx,
           graph_batch, A1_W1, A1_b1, A1_W2, A1_b2, A1_W3, A1_b3,
           A2_W1, A2_b1, A2_W2, A2_b2, A2_W3, A2_b3,
           gat_W, gat_a_src, gat_a_dst, gat_b):
    f32, i32 = jnp.float32, jnp.int32

    sei0 = supernode_edge_index[0]
    sei1 = supernode_edge_index[1]
    sei0p = jnp.concatenate([sei0, jnp.full((_EP - _E,), _N, i32)])
    sei1p = jnp.concatenate([sei1, jnp.zeros((_EP - _E,), i32)])
    gbp = jnp.concatenate(
        [graph_batch, jnp.broadcast_to(graph_batch[-1], (_GBP - _N,))])

    route = functools.partial(
        pl.kernel,
        out_type=[
            jax.ShapeDtypeStruct((_SP,), i32),
            jax.ShapeDtypeStruct((_SP,), i32),
            jax.ShapeDtypeStruct((_SP, _D), f32),
        ],
        mesh=_mesh(),
        compiler_params=pltpu.CompilerParams(needs_layout_passes=False),
        scratch_types=[
            pltpu.VMEM((_EC,), i32),
            pltpu.VMEM((_SP,), i32),
            pltpu.VMEM((_NT, _SP), i32),
            pltpu.VMEM((_COLS,), i32),
            pltpu.VMEM((_COLS,), i32),
            pltpu.VMEM((_COLS, _D), f32),
            pltpu.VMEM_SHARED((_NT, _SP), i32),
            pltpu.SemaphoreType.DMA,
        ],
    )(_route_body)
    winner, bidx, bx = route(sei0p, sei1p, x)

    x_top = lax.slice(x, (0, 0), (_SP, _D))
    nsx_p = jnp.concatenate(
        [new_supernode_x, jnp.zeros((_SP - _S, _D), f32)], axis=0)
    g = pl.pallas_call(
        _fused_body,
        out_shape=jax.ShapeDtypeStruct((_SP, _D), f32),
    )(x_top, nsx_p, bx, winner.reshape(_SP, 1), bidx.reshape(_SP, 1),
      A1_W1, A1_b1.reshape(1, -1), A1_W2, A1_b2.reshape(1, -1),
      A1_W3, A1_b3.reshape(1, -1),
      A2_W1[:_D], A2_W1[_D:], A2_b1.reshape(1, -1),
      A2_W2, A2_b2.reshape(1, -1), A2_W3, A2_b3.reshape(1, -1),
      gat_W, gat_b.reshape(1, -1))

    w_arr = lax.slice(gbp, (0,), (_GBP,), (_BLK,))
    l_arr = lax.slice(gbp, (_BLK - 1,), (_GBP,), (_BLK,))
    gb_cols = gbp.reshape(_NBLK, _BLK, 1)
    return pl.pallas_call(
        _expand_tc_body,
        grid_spec=pltpu.PrefetchScalarGridSpec(
            num_scalar_prefetch=2,
            grid=(_NBLK,),
            in_specs=[
                pl.BlockSpec((1, _BLK, 1), lambda b, *_: (b, 0, 0)),
                pl.BlockSpec((_SP, _D), lambda b, *_: (0, 0)),
            ],
            out_specs=pl.BlockSpec((_BLK, _D), lambda b, *_: (b, 0)),
        ),
        out_shape=jax.ShapeDtypeStruct((_N, _D), f32),
    )(w_arr, l_arr, gb_cols, g)


# speedup vs baseline: 1.8361x; 1.1298x over previous
import functools
import jax
import jax.numpy as jnp
from jax import lax
from jax.experimental import pallas as pl
from jax.experimental.pallas import tpu as pltpu
from jax.experimental.pallas import tpu_sc as plsc

_N = 50000
_D = 256
_S = 1000
_E = 100000

_SP = 1024
_NT = 16
_EC = 6272
_EP = _NT * _EC
_NGRP = _EC // 16
_COLS = _SP // _NT

_RC = 1568
_GBP = 32 * _RC
_SUB = 112
_NSUB = _RC // _SUB


def _mesh():
    return plsc.VectorSubcoreMesh(core_axis_name="c", subcore_axis_name="s")


def _fused_body(xt, nsx, bx, winner, bidx,
                a1w1, a1b1, a1w2, a1b2, a1w3, a1b3,
                a2w1a, a2w1b, a2b1, a2w2, a2b2, a2w3, a2b3,
                gw, gbias, out_ref):
    f32 = jnp.float32
    w1 = a1w1[...]
    h = jnp.maximum(
        jnp.dot(xt[...], w1[:_D], preferred_element_type=f32)
        + jnp.dot(nsx[...], w1[_D:], preferred_element_type=f32)
        + a1b1[...], 0.0)
    h = jnp.maximum(jnp.dot(h, a1w2[...], preferred_element_type=f32) + a1b2[...], 0.0)
    u = jnp.dot(h, a1w3[...], preferred_element_type=f32) + a1b3[...]
    rows = lax.broadcasted_iota(jnp.int32, (_SP, _D), 0)
    u = jnp.where(rows < _S, u, 0.0)

    bidxv = bidx[...]
    idx_u = jnp.minimum(bidxv, _SP - 1)
    cols = lax.broadcasted_iota(jnp.int32, (_SP, _SP), 1)
    onehot = (idx_u == cols).astype(f32)
    ug = jnp.dot(onehot, u, preferred_element_type=f32)
    b = jnp.where(bidxv < _S, ug, bx[...])
    h = jnp.maximum(
        jnp.dot(u, a2w1a[...], preferred_element_type=f32)
        + jnp.dot(b, a2w1b[...], preferred_element_type=f32)
        + a2b1[...], 0.0)
    h = jnp.maximum(jnp.dot(h, a2w2[...], preferred_element_type=f32) + a2b2[...], 0.0)
    cand = jnp.dot(h, a2w3[...], preferred_element_type=f32) + a2b3[...]
    x3 = jnp.where(winner[...] >= 0, cand, u)
    out_ref[...] = jnp.dot(x3, gw[...], preferred_element_type=f32) + gbias[...]


def _route_body(sei0_hbm, sei1_hbm, x_hbm, winner_hbm, bidx_hbm, bx_hbm,
                chunk_v, wloc_v, wall_v, wfin_v, bidx_v, rows_v, wall_sh, sem):
    c = lax.axis_index("c")
    sid = lax.axis_index("s")

    @pl.when(c == 0)
    def _core0():
        base = sid * _EC
        pltpu.sync_copy(sei0_hbm.at[pl.ds(base, _EC)], chunk_v)

        neg1 = jnp.full((16,), -1, jnp.int32)

        def _init(i, carry):
            for u in range(4):
                wloc_v[pl.ds((i * 4 + u) * 16, 16)] = neg1
            return carry

        lax.fori_loop(0, _SP // 64, _init, 0)

        lanes = lax.iota(jnp.int32, 16)

        def _scatter(g, carry):
            for u in range(4):
                off = (g * 4 + u) * 16
                idx = chunk_v[pl.ds(off, 16)]
                jv = lanes + (base + off)
                plsc.store_scatter(wloc_v, [idx], jv, mask=idx < _S)
            return carry

        lax.fori_loop(0, _NGRP // 4, _scatter, 0)

        pltpu.sync_copy(wloc_v, wall_sh.at[sid])
        plsc.subcore_barrier()
        pltpu.sync_copy(wall_sh, wall_v)

        col = sid * _COLS
        for h in range(_COLS // 16):
            acc = wall_v[0, pl.ds(col + h * 16, 16)]
            for r in range(1, _NT):
                acc = jnp.maximum(acc, wall_v[r, pl.ds(col + h * 16, 16)])
            wfin_v[pl.ds(h * 16, 16)] = acc

        pltpu.sync_copy(wfin_v, winner_hbm.at[pl.ds(col, _COLS)])

        for h in range(_COLS // 16):
            v = wfin_v[pl.ds(h * 16, 16)]
            wfin_v[pl.ds(h * 16, 16)] = jnp.maximum(v, 0)

        pltpu.async_copy(sei1_hbm.at[wfin_v], bidx_v, sem).wait()
        pltpu.sync_copy(bidx_v, bidx_hbm.at[pl.ds(col, _COLS)])
        pltpu.async_copy(x_hbm.at[bidx_v], rows_v, sem).wait()
        pltpu.sync_copy(rows_v, bx_hbm.at[pl.ds(col, _COLS)])


def _expand_body(g_hbm, gb_hbm, out_hbm, idx_v,
                 rows0, rows1, rows2, rows3, sem0, sem1, sem2, sem3):
    c = lax.axis_index("c")
    sid = lax.axis_index("s")
    wid = sid * 2 + c
    base = wid * _RC
    rows = (rows0, rows1, rows2, rows3)
    sems = (sem0, sem1, sem2, sem3)
    pltpu.sync_copy(gb_hbm.at[pl.ds(base, _RC)], idx_v)

    def _gather(k):
        return pltpu.async_copy(
            g_hbm.at[idx_v.at[pl.ds(k * _SUB, _SUB)]], rows[k % 4], sems[k % 4])

    gd = {k: _gather(k) for k in range(3)}
    sd = {}
    for k in range(_NSUB):
        gd[k].wait()
        start = base + k * _SUB
        buf = rows[k % 4]
        if k <= _NSUB - 3:
            sd[k] = pltpu.async_copy(buf, out_hbm.at[pl.ds(start, _SUB)],
                                     sems[k % 4])
        elif k == _NSUB - 2:
            @pl.when(start + _SUB <= _N)
            def _full():
                pltpu.sync_copy(buf, out_hbm.at[pl.ds(start, _SUB)])

            @pl.when(start + _SUB > _N)
            def _tail():
                for t in range(_SUB // 8):
                    @pl.when(start + t * 8 + 8 <= _N)
                    def _piece():
                        pltpu.sync_copy(
                            buf.at[pl.ds(t * 8, 8)],
                            out_hbm.at[pl.ds(start + t * 8, 8)])
        else:
            @pl.when(start + _SUB <= _N)
            def _last():
                pltpu.sync_copy(buf, out_hbm.at[pl.ds(start, _SUB)])
        if k >= 1 and (k - 1) in sd:
            sd.pop(k - 1).wait()
        if k + 3 < _NSUB:
            gd[k + 3] = _gather(k + 3)
    for k in list(sd):
        sd.pop(k).wait()


_BLK = 1024
_NBLK = _GBP // _BLK
_WINB = 64


def _expand_tc_body(w_sm, last_sm, gbv, g_ref, out_ref):
    b = pl.program_id(0)
    f32 = jnp.float32
    w = w_sm[b]
    last = last_sm[b]
    wstart = pl.multiple_of((jnp.minimum(w, _SP - _WINB) // 8) * 8, 8)
    gbb = gbv[...].reshape(_BLK, 1)
    span_ok = (last - wstart) < _WINB

    @pl.when(span_ok)
    def _fast():
        cols = lax.broadcasted_iota(jnp.int32, (_BLK, _WINB), 1)
        onehot = (gbb - wstart == cols).astype(f32)
        win = g_ref[pl.ds(wstart, _WINB), :]
        out_ref[...] = jnp.dot(onehot, win, preferred_element_type=f32)

    @pl.when(jnp.logical_not(span_ok))
    def _slow():
        cols = lax.broadcasted_iota(jnp.int32, (_BLK, _SP), 1)
        onehot = (gbb == cols).astype(f32)
        out_ref[...] = jnp.dot(onehot, g_ref[...], preferred_element_type=f32)


def kernel(x, new_supernode_x, supernode_edge_index, supernode_idx,
           graph_batch, A1_W1, A1_b1, A1_W2, A1_b2, A1_W3, A1_b3,
           A2_W1, A2_b1, A2_W2, A2_b2, A2_W3, A2_b3,
           gat_W, gat_a_src, gat_a_dst, gat_b):
    f32, i32 = jnp.float32, jnp.int32

    sei0 = supernode_edge_index[0]
    sei1 = supernode_edge_index[1]
    sei0p = jnp.concatenate([sei0, jnp.full((_EP - _E,), _N, i32)])
    sei1p = jnp.concatenate([sei1, jnp.zeros((_EP - _E,), i32)])
    gbp = jnp.concatenate(
        [graph_batch, jnp.broadcast_to(graph_batch[-1], (_GBP - _N,))])

    route = functools.partial(
        pl.kernel,
        out_type=[
            jax.ShapeDtypeStruct((_SP,), i32),
            jax.ShapeDtypeStruct((_SP,), i32),
            jax.ShapeDtypeStruct((_SP, _D), f32),
        ],
        mesh=_mesh(),
        compiler_params=pltpu.CompilerParams(needs_layout_passes=False),
        scratch_types=[
            pltpu.VMEM((_EC,), i32),
            pltpu.VMEM((_SP,), i32),
            pltpu.VMEM((_NT, _SP), i32),
            pltpu.VMEM((_COLS,), i32),
            pltpu.VMEM((_COLS,), i32),
            pltpu.VMEM((_COLS, _D), f32),
            pltpu.VMEM_SHARED((_NT, _SP), i32),
            pltpu.SemaphoreType.DMA,
        ],
    )(_route_body)
    winner, bidx, bx = route(sei0p, sei1p, x)

    x_top = lax.slice(x, (0, 0), (_SP, _D))
    nsx_p = jnp.concatenate(
        [new_supernode_x, jnp.zeros((_SP - _S, _D), f32)], axis=0)
    g = pl.pallas_call(
        _fused_body,
        out_shape=jax.ShapeDtypeStruct((_SP, _D), f32),
    )(x_top, nsx_p, bx, winner.reshape(_SP, 1), bidx.reshape(_SP, 1),
      A1_W1, A1_b1.reshape(1, -1), A1_W2, A1_b2.reshape(1, -1),
      A1_W3, A1_b3.reshape(1, -1),
      A2_W1[:_D], A2_W1[_D:], A2_b1.reshape(1, -1),
      A2_W2, A2_b2.reshape(1, -1), A2_W3, A2_b3.reshape(1, -1),
      gat_W, gat_b.reshape(1, -1))

    w_arr = lax.slice(gbp, (0,), (_GBP,), (_BLK,))
    l_arr = lax.slice(gbp, (_BLK - 1,), (_GBP,), (_BLK,))
    gb_cols = gbp.reshape(_NBLK, _BLK, 1)
    return pl.pallas_call(
        _expand_tc_body,
        grid_spec=pltpu.PrefetchScalarGridSpec(
            num_scalar_prefetch=2,
            grid=(_NBLK,),
            in_specs=[
                pl.BlockSpec((1, _BLK, 1), lambda b, *_: (b, 0, 0)),
                pl.BlockSpec((_SP, _D), lambda b, *_: (0, 0)),
            ],
            out_specs=pl.BlockSpec((_BLK, _D), lambda b, *_: (b, 0)),
        ),
        out_shape=jax.ShapeDtypeStruct((_N, _D), f32),
    )(w_arr, l_arr, gb_cols, g)


# speedup vs baseline: 2.0523x; 1.1177x over previous
import functools
import jax
import jax.numpy as jnp
from jax import lax
from jax.experimental import pallas as pl
from jax.experimental.pallas import tpu as pltpu
from jax.experimental.pallas import tpu_sc as plsc

_N = 50000
_D = 256
_S = 1000
_E = 100000

_SP = 1024
_NT = 16
_EC = 6272
_EP = _NT * _EC
_NGRP = _EC // 16
_COLS = _SP // _NT

_BLK = 2048
_NBLK = 25
_NPAD = _NBLK * _BLK
_WINB = 64


def _mesh():
    return plsc.VectorSubcoreMesh(core_axis_name="c", subcore_axis_name="s")


def _fused_body(xt, nsx, bx, winner, bidx,
                a1w1, a1b1, a1w2, a1b2, a1w3, a1b3,
                a2w1a, a2w1b, a2b1, a2w2, a2b2, a2w3, a2b3,
                gw, gbias, out_ref):
    f32 = jnp.float32
    w1 = a1w1[...]
    h = jnp.maximum(
        jnp.dot(xt[...], w1[:_D], preferred_element_type=f32)
        + jnp.dot(nsx[...], w1[_D:], preferred_element_type=f32)
        + a1b1[...], 0.0)
    h = jnp.maximum(jnp.dot(h, a1w2[...], preferred_element_type=f32) + a1b2[...], 0.0)
    u = jnp.dot(h, a1w3[...], preferred_element_type=f32) + a1b3[...]
    rows = lax.broadcasted_iota(jnp.int32, (_SP, _D), 0)
    u = jnp.where(rows < _S, u, 0.0)

    bidxv = bidx[...]
    idx_u = jnp.minimum(bidxv, _SP - 1)
    cols = lax.broadcasted_iota(jnp.int32, (_SP, _SP), 1)
    onehot = (idx_u == cols).astype(f32)
    ug = jnp.dot(onehot, u, preferred_element_type=f32)
    b = jnp.where(bidxv < _S, ug, bx[...])
    h = jnp.maximum(
        jnp.dot(u, a2w1a[...], preferred_element_type=f32)
        + jnp.dot(b, a2w1b[...], preferred_element_type=f32)
        + a2b1[...], 0.0)
    h = jnp.maximum(jnp.dot(h, a2w2[...], preferred_element_type=f32) + a2b2[...], 0.0)
    cand = jnp.dot(h, a2w3[...], preferred_element_type=f32) + a2b3[...]
    x3 = jnp.where(winner[...] >= 0, cand, u)
    out_ref[...] = jnp.dot(x3, gw[...], preferred_element_type=f32) + gbias[...]


def _route_body(sei0_hbm, sei1_hbm, x_hbm, winner_hbm, bidx_hbm, bx_hbm,
                chunk_v, wloc_v, wall_v, wfin_v, bidx_v, rows_v, wall_sh, sem):
    c = lax.axis_index("c")
    sid = lax.axis_index("s")

    @pl.when(c == 0)
    def _core0():
        base = sid * _EC
        pltpu.sync_copy(sei0_hbm.at[pl.ds(base, _EC)], chunk_v)

        neg1 = jnp.full((16,), -1, jnp.int32)

        def _init(i, carry):
            for u in range(4):
                wloc_v[pl.ds((i * 4 + u) * 16, 16)] = neg1
            return carry

        lax.fori_loop(0, _SP // 64, _init, 0)

        lanes = lax.iota(jnp.int32, 16)

        def _scatter(g, carry):
            for u in range(4):
                off = (g * 4 + u) * 16
                idx = chunk_v[pl.ds(off, 16)]
                jv = lanes + (base + off)
                plsc.store_scatter(wloc_v, [idx], jv, mask=idx < _S)
            return carry

        lax.fori_loop(0, _NGRP // 4, _scatter, 0)

        pltpu.sync_copy(wloc_v, wall_sh.at[sid])
        plsc.subcore_barrier()
        pltpu.sync_copy(wall_sh, wall_v)

        col = sid * _COLS
        for h in range(_COLS // 16):
            acc = wall_v[0, pl.ds(col + h * 16, 16)]
            for r in range(1, _NT):
                acc = jnp.maximum(acc, wall_v[r, pl.ds(col + h * 16, 16)])
            wfin_v[pl.ds(h * 16, 16)] = acc

        pltpu.sync_copy(wfin_v, winner_hbm.at[pl.ds(col, _COLS)])

        for h in range(_COLS // 16):
            v = wfin_v[pl.ds(h * 16, 16)]
            wfin_v[pl.ds(h * 16, 16)] = jnp.maximum(v, 0)

        pltpu.async_copy(sei1_hbm.at[wfin_v], bidx_v, sem).wait()
        pltpu.sync_copy(bidx_v, bidx_hbm.at[pl.ds(col, _COLS)])
        pltpu.async_copy(x_hbm.at[bidx_v], rows_v, sem).wait()
        pltpu.sync_copy(rows_v, bx_hbm.at[pl.ds(col, _COLS)])


def _expand_tc_body(w_sm, last_sm, gbv, g_ref, out_ref):
    b = pl.program_id(0)
    f32 = jnp.float32
    w = w_sm[b]
    last = last_sm[b]
    wstart = pl.multiple_of((jnp.minimum(w, _SP - _WINB) // 8) * 8, 8)
    gbb = gbv[...].reshape(_BLK, 1)
    span_ok = (last - wstart) < _WINB

    @pl.when(span_ok)
    def _fast():
        cols = lax.broadcasted_iota(jnp.int32, (_BLK, _WINB), 1)
        onehot = (gbb - wstart == cols).astype(f32)
        win = g_ref[pl.ds(wstart, _WINB), :]
        out_ref[...] = jnp.dot(onehot, win, preferred_element_type=f32)

    @pl.when(jnp.logical_not(span_ok))
    def _slow():
        cols = lax.broadcasted_iota(jnp.int32, (_BLK, _SP), 1)
        onehot = (gbb == cols).astype(f32)
        out_ref[...] = jnp.dot(onehot, g_ref[...], preferred_element_type=f32)


def kernel(x, new_supernode_x, supernode_edge_index, supernode_idx,
           graph_batch, A1_W1, A1_b1, A1_W2, A1_b2, A1_W3, A1_b3,
           A2_W1, A2_b1, A2_W2, A2_b2, A2_W3, A2_b3,
           gat_W, gat_a_src, gat_a_dst, gat_b):
    f32, i32 = jnp.float32, jnp.int32

    sei0 = supernode_edge_index[0]
    sei1 = supernode_edge_index[1]
    sei0p = jnp.concatenate([sei0, jnp.full((_EP - _E,), _N, i32)])
    sei1p = jnp.concatenate([sei1, jnp.zeros((_EP - _E,), i32)])
    gbp = jnp.concatenate(
        [graph_batch, jnp.broadcast_to(graph_batch[-1], (_NPAD - _N,))])

    route = functools.partial(
        pl.kernel,
        out_type=[
            jax.ShapeDtypeStruct((_SP,), i32),
            jax.ShapeDtypeStruct((_SP,), i32),
            jax.ShapeDtypeStruct((_SP, _D), f32),
        ],
        mesh=_mesh(),
        compiler_params=pltpu.CompilerParams(needs_layout_passes=False),
        scratch_types=[
            pltpu.VMEM((_EC,), i32),
            pltpu.VMEM((_SP,), i32),
            pltpu.VMEM((_NT, _SP), i32),
            pltpu.VMEM((_COLS,), i32),
            pltpu.VMEM((_COLS,), i32),
            pltpu.VMEM((_COLS, _D), f32),
            pltpu.VMEM_SHARED((_NT, _SP), i32),
            pltpu.SemaphoreType.DMA,
        ],
    )(_route_body)
    winner, bidx, bx = route(sei0p, sei1p, x)

    x_top = lax.slice(x, (0, 0), (_SP, _D))
    nsx_p = jnp.concatenate(
        [new_supernode_x, jnp.zeros((_SP - _S, _D), f32)], axis=0)
    g = pl.pallas_call(
        _fused_body,
        out_shape=jax.ShapeDtypeStruct((_SP, _D), f32),
    )(x_top, nsx_p, bx, winner.reshape(_SP, 1), bidx.reshape(_SP, 1),
      A1_W1, A1_b1.reshape(1, -1), A1_W2, A1_b2.reshape(1, -1),
      A1_W3, A1_b3.reshape(1, -1),
      A2_W1[:_D], A2_W1[_D:], A2_b1.reshape(1, -1),
      A2_W2, A2_b2.reshape(1, -1), A2_W3, A2_b3.reshape(1, -1),
      gat_W, gat_b.reshape(1, -1))

    w_arr = lax.slice(gbp, (0,), (_NPAD,), (_BLK,))
    l_arr = lax.slice(gbp, (_BLK - 1,), (_NPAD,), (_BLK,))
    gb_cols = gbp.reshape(_NBLK, _BLK, 1)
    return pl.pallas_call(
        _expand_tc_body,
        grid_spec=pltpu.PrefetchScalarGridSpec(
            num_scalar_prefetch=2,
            grid=(_NBLK,),
            in_specs=[
                pl.BlockSpec((1, _BLK, 1), lambda b, *_: (b, 0, 0)),
                pl.BlockSpec((_SP, _D), lambda b, *_: (0, 0)),
            ],
            out_specs=pl.BlockSpec((_BLK, _D), lambda b, *_: (b, 0)),
        ),
        out_shape=jax.ShapeDtypeStruct((_N, _D), f32),
    )(w_arr, l_arr, gb_cols, g)


# speedup vs baseline: 2.2137x; 1.0787x over previous
import functools
import jax
import jax.numpy as jnp
from jax import lax
from jax.experimental import pallas as pl
from jax.experimental.pallas import tpu as pltpu
from jax.experimental.pallas import tpu_sc as plsc

_N = 50000
_D = 256
_S = 1000
_E = 100000

_SP = 1024
_NT = 16
_EC = 6272
_EP = _NT * _EC
_NGRP = _EC // 16
_COLS = _SP // _NT

_BLK = 4096
_NBLK = 13
_NPAD = _NBLK * _BLK
_WINB = 128


def _mesh():
    return plsc.VectorSubcoreMesh(core_axis_name="c", subcore_axis_name="s")


def _fused_body(xt, nsx, bx, winner, bidx,
                a1w1, a1b1, a1w2, a1b2, a1w3, a1b3,
                a2w1a, a2w1b, a2b1, a2w2, a2b2, a2w3, a2b3,
                gw, gbias, out_ref):
    f32 = jnp.float32
    w1 = a1w1[...]
    h = jnp.maximum(
        jnp.dot(xt[...], w1[:_D], preferred_element_type=f32)
        + jnp.dot(nsx[...], w1[_D:], preferred_element_type=f32)
        + a1b1[...], 0.0)
    h = jnp.maximum(jnp.dot(h, a1w2[...], preferred_element_type=f32) + a1b2[...], 0.0)
    u = jnp.dot(h, a1w3[...], preferred_element_type=f32) + a1b3[...]
    rows = lax.broadcasted_iota(jnp.int32, (_SP, _D), 0)
    u = jnp.where(rows < _S, u, 0.0)

    bidxv = bidx[...]
    idx_u = jnp.minimum(bidxv, _SP - 1)
    cols = lax.broadcasted_iota(jnp.int32, (_SP, _SP), 1)
    onehot = (idx_u == cols).astype(f32)
    ug = jnp.dot(onehot, u, preferred_element_type=f32)
    b = jnp.where(bidxv < _S, ug, bx[...])
    h = jnp.maximum(
        jnp.dot(u, a2w1a[...], preferred_element_type=f32)
        + jnp.dot(b, a2w1b[...], preferred_element_type=f32)
        + a2b1[...], 0.0)
    h = jnp.maximum(jnp.dot(h, a2w2[...], preferred_element_type=f32) + a2b2[...], 0.0)
    cand = jnp.dot(h, a2w3[...], preferred_element_type=f32) + a2b3[...]
    x3 = jnp.where(winner[...] >= 0, cand, u)
    out_ref[...] = jnp.dot(x3, gw[...], preferred_element_type=f32) + gbias[...]


def _route_body(sei0_hbm, sei1_hbm, x_hbm, winner_hbm, bidx_hbm, bx_hbm,
                chunk_v, wloc_v, wall_v, wfin_v, bidx_v, rows_v, wall_sh, sem):
    c = lax.axis_index("c")
    sid = lax.axis_index("s")

    @pl.when(c == 0)
    def _core0():
        base = sid * _EC
        pltpu.sync_copy(sei0_hbm.at[pl.ds(base, _EC)], chunk_v)

        neg1 = jnp.full((16,), -1, jnp.int32)

        def _init(i, carry):
            for u in range(4):
                wloc_v[pl.ds((i * 4 + u) * 16, 16)] = neg1
            return carry

        lax.fori_loop(0, _SP // 64, _init, 0)

        lanes = lax.iota(jnp.int32, 16)

        def _scatter(g, carry):
            for u in range(4):
                off = (g * 4 + u) * 16
                idx = chunk_v[pl.ds(off, 16)]
                jv = lanes + (base + off)
                plsc.store_scatter(wloc_v, [idx], jv, mask=idx < _S)
            return carry

        lax.fori_loop(0, _NGRP // 4, _scatter, 0)

        pltpu.sync_copy(wloc_v, wall_sh.at[sid])
        plsc.subcore_barrier()
        pltpu.sync_copy(wall_sh, wall_v)

        col = sid * _COLS
        for h in range(_COLS // 16):
            acc = wall_v[0, pl.ds(col + h * 16, 16)]
            for r in range(1, _NT):
                acc = jnp.maximum(acc, wall_v[r, pl.ds(col + h * 16, 16)])
            wfin_v[pl.ds(h * 16, 16)] = acc

        pltpu.sync_copy(wfin_v, winner_hbm.at[pl.ds(col, _COLS)])

        for h in range(_COLS // 16):
            v = wfin_v[pl.ds(h * 16, 16)]
            wfin_v[pl.ds(h * 16, 16)] = jnp.maximum(v, 0)

        pltpu.async_copy(sei1_hbm.at[wfin_v], bidx_v, sem).wait()
        pltpu.sync_copy(bidx_v, bidx_hbm.at[pl.ds(col, _COLS)])
        pltpu.async_copy(x_hbm.at[bidx_v], rows_v, sem).wait()
        pltpu.sync_copy(rows_v, bx_hbm.at[pl.ds(col, _COLS)])


def _expand_tc_body(gbv, g_ref, out_ref):
    f32 = jnp.float32
    gbb = gbv[...].reshape(_BLK, 1)
    w = jnp.min(gbb)
    last = jnp.max(gbb)
    wstart = pl.multiple_of((jnp.minimum(w, _SP - _WINB) // 8) * 8, 8)
    span_ok = (last - wstart) < _WINB

    @pl.when(span_ok)
    def _fast():
        cols = lax.broadcasted_iota(jnp.int32, (_BLK, _WINB), 1)
        onehot = (gbb - wstart == cols).astype(f32)
        win = g_ref[pl.ds(wstart, _WINB), :]
        out_ref[...] = jnp.dot(onehot, win, preferred_element_type=f32)

    @pl.when(jnp.logical_not(span_ok))
    def _slow():
        cols = lax.broadcasted_iota(jnp.int32, (_BLK, _SP), 1)
        onehot = (gbb == cols).astype(f32)
        out_ref[...] = jnp.dot(onehot, g_ref[...], preferred_element_type=f32)


def kernel(x, new_supernode_x, supernode_edge_index, supernode_idx,
           graph_batch, A1_W1, A1_b1, A1_W2, A1_b2, A1_W3, A1_b3,
           A2_W1, A2_b1, A2_W2, A2_b2, A2_W3, A2_b3,
           gat_W, gat_a_src, gat_a_dst, gat_b):
    f32, i32 = jnp.float32, jnp.int32

    sei0 = supernode_edge_index[0]
    sei1 = supernode_edge_index[1]
    sei0p = jnp.concatenate([sei0, jnp.full((_EP - _E,), _N, i32)])
    sei1p = jnp.concatenate([sei1, jnp.zeros((_EP - _E,), i32)])
    gbp = jnp.concatenate(
        [graph_batch, jnp.broadcast_to(graph_batch[-1], (_NPAD - _N,))])

    route = functools.partial(
        pl.kernel,
        out_type=[
            jax.ShapeDtypeStruct((_SP,), i32),
            jax.ShapeDtypeStruct((_SP,), i32),
            jax.ShapeDtypeStruct((_SP, _D), f32),
        ],
        mesh=_mesh(),
        compiler_params=pltpu.CompilerParams(needs_layout_passes=False),
        scratch_types=[
            pltpu.VMEM((_EC,), i32),
            pltpu.VMEM((_SP,), i32),
            pltpu.VMEM((_NT, _SP), i32),
            pltpu.VMEM((_COLS,), i32),
            pltpu.VMEM((_COLS,), i32),
            pltpu.VMEM((_COLS, _D), f32),
            pltpu.VMEM_SHARED((_NT, _SP), i32),
            pltpu.SemaphoreType.DMA,
        ],
    )(_route_body)
    winner, bidx, bx = route(sei0p, sei1p, x)

    x_top = lax.slice(x, (0, 0), (_SP, _D))
    nsx_p = jnp.concatenate(
        [new_supernode_x, jnp.zeros((_SP - _S, _D), f32)], axis=0)
    g = pl.pallas_call(
        _fused_body,
        out_shape=jax.ShapeDtypeStruct((_SP, _D), f32),
    )(x_top, nsx_p, bx, winner.reshape(_SP, 1), bidx.reshape(_SP, 1),
      A1_W1, A1_b1.reshape(1, -1), A1_W2, A1_b2.reshape(1, -1),
      A1_W3, A1_b3.reshape(1, -1),
      A2_W1[:_D], A2_W1[_D:], A2_b1.reshape(1, -1),
      A2_W2, A2_b2.reshape(1, -1), A2_W3, A2_b3.reshape(1, -1),
      gat_W, gat_b.reshape(1, -1))

    gb_cols = gbp.reshape(_NBLK, _BLK, 1)
    return pl.pallas_call(
        _expand_tc_body,
        grid=(_NBLK,),
        in_specs=[
            pl.BlockSpec((1, _BLK, 1), lambda b: (b, 0, 0)),
            pl.BlockSpec((_SP, _D), lambda b: (0, 0)),
        ],
        out_specs=pl.BlockSpec((_BLK, _D), lambda b: (b, 0)),
        out_shape=jax.ShapeDtypeStruct((_N, _D), f32),
    )(gb_cols, g)
